# baseline (speedup 1.0000x reference)
import jax
import jax.numpy as jnp
from jax import lax
from jax.experimental import pallas as pl
from jax.experimental.pallas import tpu as pltpu
from jax.experimental.pallas import tpu_sc as plsc

N = 10000
E = 320000
D = 128
ALPHA = 0.2

NC = 2
NS = 16
L = 16
NW = NC * NS
C = 64
NCH = E // C
BASE_PAIRS = NCH // 2 // NW
EXTRA = NCH // 2 - BASE_PAIRS * NW
N_PAD = 10112
RPW = N_PAD // NS


def _dense_body(h_ref, w_ref, a2_ref, wh_ref, wh12_ref):
    wh = jnp.dot(h_ref[...], w_ref[...],
                 preferred_element_type=jnp.float32,
                 precision=lax.Precision.HIGHEST)
    wh_ref[...] = wh
    wh12_ref[...] = lax.dot_general(
        a2_ref[...], wh, (((0,), (1,)), ((), ())),
        preferred_element_type=jnp.float32,
        precision=lax.Precision.HIGHEST)


def _dense(h, W, a2):
    return pl.pallas_call(
        _dense_body,
        out_shape=[
            jax.ShapeDtypeStruct((N, D), jnp.float32),
            jax.ShapeDtypeStruct((2, N), jnp.float32),
        ],
    )(h, W, a2)


def _edge_body(wh_hbm, wh12_hbm, erow_hbm, ecol_hbm, unnorm_hbm, rowsum_hbm,
               wh12_v, rs_v, v_v, rowe_v, cole_v, ridx_a, ridx_b,
               rows_a, rows_b, unnorm_sh,
               gsem_a, gsem_b, esem):
    cid = lax.axis_index("c")
    sid = lax.axis_index("s")
    wid = sid * NC + cid

    zeros = jnp.zeros((L,), jnp.float32)
    zero16i = jnp.zeros((L,), jnp.int32)
    one16i = jnp.ones((L,), jnp.int32)

    npairs = BASE_PAIRS + jnp.where(wid < EXTRA, 1, 0)
    start = 2 * (BASE_PAIRS * wid + lax.min(wid, EXTRA))

    @pl.loop(0, N_PAD // L)
    def _(i):
        rs_v[pl.ds(i * L, L)] = zeros

    @pl.loop(0, C)
    def _(i):
        for j in range(D // L):
            rows_a[i, pl.ds(j * L, L)] = zeros

    for k in range(RPW // C):
        pltpu.sync_copy(rows_a, unnorm_sh.at[pl.ds(sid * RPW + k * C, C)])
    pltpu.sync_copy(rows_a.at[pl.ds(0, RPW - (RPW // C) * C)],
                    unnorm_sh.at[pl.ds(sid * RPW + (RPW // C) * C,
                                       RPW - (RPW // C) * C)])

    pltpu.sync_copy(wh12_hbm, wh12_v)
    pltpu.sync_copy(erow_hbm.at[pl.ds(start * C, 2 * C)], rowe_v)
    pltpu.sync_copy(ecol_hbm.at[pl.ds(start * C, 2 * C)], cole_v)
    pltpu.async_copy(wh_hbm.at[cole_v.at[pl.ds(0, C)]], rows_a, gsem_a)
    pltpu.async_copy(wh_hbm.at[cole_v.at[pl.ds(C, C)]], rows_b, gsem_b)

    plsc.subcore_barrier()

    def compute_v_and_scale(j, ridx, rows):
        for g in range(C // L):
            r16 = rowe_v[pl.ds(j * C + g * L, L)]
            c16 = cole_v[pl.ds(j * C + g * L, L)]
            ridx[pl.ds(g * L, L)] = r16
            ee = (plsc.load_gather(wh12_v, [zero16i, r16])
                  + plsc.load_gather(wh12_v, [one16i, c16]))
            ee = jnp.where(ee >= 0, ee, ALPHA * ee)
            vv = jnp.exp(ee)
            v_v[pl.ds(g * L, L)] = vv
            if g == 0:
                v_v[pl.ds(C, L)] = vv
            plsc.addupdate_scatter(rs_v, [r16], vv)
        for i in range(C):
            b = plsc.load_gather(
                v_v, [jnp.full((L,), i if i else C, jnp.int32)])
            for k in range(D // L):
                sl = (i, pl.ds(k * L, L))
                rows[sl] = rows[sl] * b

    @pl.loop(0, npairs)
    def _(i):
        c = start + 2 * i
        not_last = i < npairs - 1

        pltpu.make_async_copy(wh_hbm.at[cole_v.at[pl.ds(0, C)]],
                              rows_a, gsem_a).wait()
        compute_v_and_scale(0, ridx_a, rows_a)
        pltpu.sync_copy(rows_a, unnorm_sh.at[ridx_a], add=True)

        pltpu.make_async_copy(wh_hbm.at[cole_v.at[pl.ds(C, C)]],
                              rows_b, gsem_b).wait()
        compute_v_and_scale(1, ridx_b, rows_b)

        @pl.when(not_last)
        def _():
            pltpu.async_copy(erow_hbm.at[pl.ds((c + 2) * C, 2 * C)],
                             rowe_v, esem)
            pltpu.async_copy(ecol_hbm.at[pl.ds((c + 2) * C, 2 * C)],
                             cole_v, esem)

        pltpu.sync_copy(rows_b, unnorm_sh.at[ridx_b], add=True)

        @pl.when(not_last)
        def _():
            pltpu.make_async_copy(erow_hbm.at[pl.ds((c + 2) * C, 2 * C)],
                                  rowe_v, esem).wait()
            pltpu.make_async_copy(ecol_hbm.at[pl.ds((c + 2) * C, 2 * C)],
                                  cole_v, esem).wait()
            pltpu.async_copy(wh_hbm.at[cole_v.at[pl.ds(0, C)]],
                             rows_a, gsem_a)
            pltpu.async_copy(wh_hbm.at[cole_v.at[pl.ds(C, C)]],
                             rows_b, gsem_b)

    plsc.subcore_barrier()

    pltpu.sync_copy(unnorm_sh.at[pl.ds(sid * RPW, RPW)],
                    unnorm_hbm.at[cid, pl.ds(sid * RPW, RPW)])
    pltpu.sync_copy(rs_v, rowsum_hbm.at[pl.ds(wid * N_PAD, N_PAD)])


def _edge_pass(wh, wh12, erow, ecol):
    mesh = plsc.VectorSubcoreMesh(core_axis_name="c", subcore_axis_name="s")
    kern = pl.kernel(
        _edge_body,
        out_type=[
            jax.ShapeDtypeStruct((NC, N_PAD, D), jnp.float32),
            jax.ShapeDtypeStruct((NW * N_PAD,), jnp.float32),
        ],
        mesh=mesh,
        compiler_params=pltpu.CompilerParams(needs_layout_passes=False),
        scratch_types=[
            pltpu.VMEM((2, N), jnp.float32),
            pltpu.VMEM((N_PAD,), jnp.float32),
            pltpu.VMEM((C + L,), jnp.float32),
            pltpu.VMEM((2 * C,), jnp.int32),
            pltpu.VMEM((2 * C,), jnp.int32),
            pltpu.VMEM((C,), jnp.int32),
            pltpu.VMEM((C,), jnp.int32),
            pltpu.VMEM((C, D), jnp.float32),
            pltpu.VMEM((C, D), jnp.float32),
            pltpu.VMEM_SHARED((N_PAD, D), jnp.float32),
            pltpu.SemaphoreType.DMA,
            pltpu.SemaphoreType.DMA,
            pltpu.SemaphoreType.DMA,
        ],
    )
    return kern(wh, wh12, erow, ecol)


def _combine_body(u_ref, rs_ref, out_ref):
    rs = jnp.sum(rs_ref[...], axis=0)
    u = u_ref[0] + u_ref[1]
    rs_col = rs[:, None]
    safe = jnp.where(rs_col > 0, rs_col, 1.0)
    x = jnp.where(rs_col > 0, u / safe, 0.0)
    out_ref[...] = jnp.where(x > 0, x, jnp.exp(jnp.minimum(x, 0.0)) - 1.0)


def _combine(unnorm, rowsum2d):
    return pl.pallas_call(
        _combine_body,
        out_shape=jax.ShapeDtypeStruct((N_PAD, D), jnp.float32),
    )(unnorm, rowsum2d)


def kernel(h, edge_index, W, a):
    a2 = jnp.concatenate([a[:D], a[D:]], axis=1)
    wh, wh12 = _dense(h, W, a2)
    unnorm, rowsum = _edge_pass(wh, wh12, edge_index[0], edge_index[1])
    out = _combine(unnorm, rowsum.reshape(NW, N_PAD))
    return out[:N]


# speedup vs baseline: 1.5982x; 1.5982x over previous
import jax
import jax.numpy as jnp
from jax import lax
from jax.experimental import pallas as pl
from jax.experimental.pallas import tpu as pltpu
from jax.experimental.pallas import tpu_sc as plsc

N = 10000
E = 320000
D = 128
ALPHA = 0.2

NC = 2
NS = 16
L = 16
NW = NC * NS
C = 64
NCH = E // C
BASE_PAIRS = NCH // 2 // NW
EXTRA = NCH // 2 - BASE_PAIRS * NW
N_PAD = 10112
RPW = N_PAD // NS


def _dense_body(h_ref, w_ref, a2_ref, wh_ref, wh12_ref):
    wh = jnp.dot(h_ref[...], w_ref[...],
                 preferred_element_type=jnp.float32,
                 precision=lax.Precision.HIGHEST)
    wh_ref[...] = wh
    wh12_ref[...] = lax.dot_general(
        a2_ref[...], wh, (((0,), (1,)), ((), ())),
        preferred_element_type=jnp.float32,
        precision=lax.Precision.HIGHEST)


def _dense(h, W, a2):
    return pl.pallas_call(
        _dense_body,
        out_shape=[
            jax.ShapeDtypeStruct((N, D), jnp.float32),
            jax.ShapeDtypeStruct((2, N), jnp.float32),
        ],
    )(h, W, a2)


def _edge_body(wh_hbm, wh12_hbm, erow_hbm, ecol_hbm, unnorm_hbm, rowsum_hbm,
               wh12_v, rs_v, v_v, rowe_v, cole_v, ridx_a, ridx_b,
               rows_a, rows_b, unnorm_sh,
               gsem_a, gsem_b, esem):
    cid = lax.axis_index("c")
    sid = lax.axis_index("s")
    wid = sid * NC + cid

    zeros = jnp.zeros((L,), jnp.float32)
    zero16i = jnp.zeros((L,), jnp.int32)
    one16i = jnp.ones((L,), jnp.int32)

    npairs = BASE_PAIRS + jnp.where(wid < EXTRA, 1, 0)
    start = 2 * (BASE_PAIRS * wid + lax.min(wid, EXTRA))

    @pl.loop(0, N_PAD // L)
    def _(i):
        rs_v[pl.ds(i * L, L)] = zeros

    @pl.loop(0, C)
    def _(i):
        for j in range(D // L):
            rows_a[i, pl.ds(j * L, L)] = zeros

    for k in range(RPW // C):
        pltpu.sync_copy(rows_a, unnorm_sh.at[pl.ds(sid * RPW + k * C, C)])
    pltpu.sync_copy(rows_a.at[pl.ds(0, RPW - (RPW // C) * C)],
                    unnorm_sh.at[pl.ds(sid * RPW + (RPW // C) * C,
                                       RPW - (RPW // C) * C)])

    pltpu.sync_copy(wh12_hbm, wh12_v)
    pltpu.sync_copy(erow_hbm.at[pl.ds(start * C, 2 * C)], rowe_v)
    pltpu.sync_copy(ecol_hbm.at[pl.ds(start * C, 2 * C)], cole_v)
    pltpu.async_copy(wh_hbm.at[cole_v.at[pl.ds(0, C)]], rows_a, gsem_a)
    pltpu.async_copy(wh_hbm.at[cole_v.at[pl.ds(C, C)]], rows_b, gsem_b)

    plsc.subcore_barrier()

    def compute_v_and_scale(j, ridx, rows):
        for g in range(C // L):
            r16 = rowe_v[pl.ds(j * C + g * L, L)]
            c16 = cole_v[pl.ds(j * C + g * L, L)]
            ridx[pl.ds(g * L, L)] = r16
            ee = (plsc.load_gather(wh12_v, [zero16i, r16])
                  + plsc.load_gather(wh12_v, [one16i, c16]))
            ee = jnp.where(ee >= 0, ee, ALPHA * ee)
            vv = jnp.exp(ee)
            v_v[pl.ds(g * L, L)] = vv
            plsc.addupdate_scatter(rs_v, [r16], vv)

        @plsc.parallel_loop(0, C, 1, unroll=8)
        def _(i):
            b = plsc.load_gather(v_v, [jnp.zeros((L,), jnp.int32) + i])
            for k in range(D // L):
                sl = (i, pl.ds(k * L, L))
                rows[sl] = rows[sl] * b

    @pl.loop(0, npairs)
    def _(i):
        c = start + 2 * i
        not_last = i < npairs - 1

        pltpu.make_async_copy(wh_hbm.at[cole_v.at[pl.ds(0, C)]],
                              rows_a, gsem_a).wait()
        compute_v_and_scale(0, ridx_a, rows_a)
        pltpu.sync_copy(rows_a, unnorm_sh.at[ridx_a], add=True)

        pltpu.make_async_copy(wh_hbm.at[cole_v.at[pl.ds(C, C)]],
                              rows_b, gsem_b).wait()
        compute_v_and_scale(1, ridx_b, rows_b)

        @pl.when(not_last)
        def _():
            pltpu.async_copy(erow_hbm.at[pl.ds((c + 2) * C, 2 * C)],
                             rowe_v, esem)
            pltpu.async_copy(ecol_hbm.at[pl.ds((c + 2) * C, 2 * C)],
                             cole_v, esem)

        pltpu.sync_copy(rows_b, unnorm_sh.at[ridx_b], add=True)

        @pl.when(not_last)
        def _():
            pltpu.make_async_copy(erow_hbm.at[pl.ds((c + 2) * C, 2 * C)],
                                  rowe_v, esem).wait()
            pltpu.make_async_copy(ecol_hbm.at[pl.ds((c + 2) * C, 2 * C)],
                                  cole_v, esem).wait()
            pltpu.async_copy(wh_hbm.at[cole_v.at[pl.ds(0, C)]],
                             rows_a, gsem_a)
            pltpu.async_copy(wh_hbm.at[cole_v.at[pl.ds(C, C)]],
                             rows_b, gsem_b)

    plsc.subcore_barrier()

    pltpu.sync_copy(unnorm_sh.at[pl.ds(sid * RPW, RPW)],
                    unnorm_hbm.at[cid, pl.ds(sid * RPW, RPW)])
    pltpu.sync_copy(rs_v, rowsum_hbm.at[pl.ds(wid * N_PAD, N_PAD)])


def _edge_pass(wh, wh12, erow, ecol):
    mesh = plsc.VectorSubcoreMesh(core_axis_name="c", subcore_axis_name="s")
    kern = pl.kernel(
        _edge_body,
        out_type=[
            jax.ShapeDtypeStruct((NC, N_PAD, D), jnp.float32),
            jax.ShapeDtypeStruct((NW * N_PAD,), jnp.float32),
        ],
        mesh=mesh,
        compiler_params=pltpu.CompilerParams(needs_layout_passes=False),
        scratch_types=[
            pltpu.VMEM((2, N), jnp.float32),
            pltpu.VMEM((N_PAD,), jnp.float32),
            pltpu.VMEM((C + L,), jnp.float32),
            pltpu.VMEM((2 * C,), jnp.int32),
            pltpu.VMEM((2 * C,), jnp.int32),
            pltpu.VMEM((C,), jnp.int32),
            pltpu.VMEM((C,), jnp.int32),
            pltpu.VMEM((C, D), jnp.float32),
            pltpu.VMEM((C, D), jnp.float32),
            pltpu.VMEM_SHARED((N_PAD, D), jnp.float32),
            pltpu.SemaphoreType.DMA,
            pltpu.SemaphoreType.DMA,
            pltpu.SemaphoreType.DMA,
        ],
    )
    return kern(wh, wh12, erow, ecol)


def _combine_body(u_ref, rs_ref, out_ref):
    rs = jnp.sum(rs_ref[...], axis=0)
    u = u_ref[0] + u_ref[1]
    rs_col = rs[:, None]
    safe = jnp.where(rs_col > 0, rs_col, 1.0)
    x = jnp.where(rs_col > 0, u / safe, 0.0)
    out_ref[...] = jnp.where(x > 0, x, jnp.exp(jnp.minimum(x, 0.0)) - 1.0)


def _combine(unnorm, rowsum2d):
    return pl.pallas_call(
        _combine_body,
        out_shape=jax.ShapeDtypeStruct((N_PAD, D), jnp.float32),
    )(unnorm, rowsum2d)


def kernel(h, edge_index, W, a):
    a2 = jnp.concatenate([a[:D], a[D:]], axis=1)
    wh, wh12 = _dense(h, W, a2)
    unnorm, rowsum = _edge_pass(wh, wh12, edge_index[0], edge_index[1])
    out = _combine(unnorm, rowsum.reshape(NW, N_PAD))
    return out[:N]


# speedup vs baseline: 2.0979x; 1.3127x over previous
import jax
import jax.numpy as jnp
from jax import lax
from jax.experimental import pallas as pl
from jax.experimental.pallas import tpu as pltpu
from jax.experimental.pallas import tpu_sc as plsc

N = 10000
E = 320000
D = 128
ALPHA = 0.2

NC = 2
NS = 16
L = 16
NW = NC * NS
C = 64
NCH = E // C
BASE_PAIRS = NCH // 2 // NW
EXTRA = NCH // 2 - BASE_PAIRS * NW
N_PAD = 10112
RPW = N_PAD // NS


def _dense_body(h_ref, w_ref, a2_ref, wh_ref, wh12_ref):
    wh = jnp.dot(h_ref[...], w_ref[...],
                 preferred_element_type=jnp.float32,
                 precision=lax.Precision.HIGHEST)
    wh_ref[...] = wh
    wh12_ref[...] = lax.dot_general(
        a2_ref[...], wh, (((0,), (1,)), ((), ())),
        preferred_element_type=jnp.float32,
        precision=lax.Precision.HIGHEST)


def _dense(h, W, a2):
    return pl.pallas_call(
        _dense_body,
        out_shape=[
            jax.ShapeDtypeStruct((N, D), jnp.float32),
            jax.ShapeDtypeStruct((2, N), jnp.float32),
        ],
    )(h, W, a2)


def _edge_body(wh_hbm, wh12_hbm, erow_hbm, ecol_hbm, unnorm_hbm, rowsum_hbm,
               wh12_v, rs_v, v_v, rowe_a, cole_a, rowe_b, cole_b,
               ridx_a, ridx_b, rows_a, rows_b, unnorm_sh,
               gsem_a, gsem_b, esem_a, esem_b):
    cid = lax.axis_index("c")
    sid = lax.axis_index("s")
    wid = sid * NC + cid

    zeros = jnp.zeros((L,), jnp.float32)
    zero16i = jnp.zeros((L,), jnp.int32)
    one16i = jnp.ones((L,), jnp.int32)

    npairs = BASE_PAIRS + jnp.where(wid < EXTRA, 1, 0)
    start = 2 * (BASE_PAIRS * wid + lax.min(wid, EXTRA))

    @pl.loop(0, N_PAD // L)
    def _(i):
        rs_v[pl.ds(i * L, L)] = zeros

    @pl.loop(0, C)
    def _(i):
        for j in range(D // L):
            rows_a[i, pl.ds(j * L, L)] = zeros

    for k in range(RPW // C):
        pltpu.sync_copy(rows_a, unnorm_sh.at[pl.ds(sid * RPW + k * C, C)])
    pltpu.sync_copy(rows_a.at[pl.ds(0, RPW - (RPW // C) * C)],
                    unnorm_sh.at[pl.ds(sid * RPW + (RPW // C) * C,
                                       RPW - (RPW // C) * C)])

    pltpu.sync_copy(wh12_hbm, wh12_v)
    pltpu.sync_copy(erow_hbm.at[pl.ds(start * C, C)], rowe_a)
    pltpu.sync_copy(ecol_hbm.at[pl.ds(start * C, C)], cole_a)
    pltpu.sync_copy(erow_hbm.at[pl.ds((start + 1) * C, C)], rowe_b)
    pltpu.sync_copy(ecol_hbm.at[pl.ds((start + 1) * C, C)], cole_b)
    pltpu.async_copy(wh_hbm.at[cole_a], rows_a, gsem_a)
    pltpu.async_copy(wh_hbm.at[cole_b], rows_b, gsem_b)

    plsc.subcore_barrier()

    def compute_v_and_scale(rowe, cole, ridx, rows):
        for g in range(C // L):
            r16 = rowe[pl.ds(g * L, L)]
            c16 = cole[pl.ds(g * L, L)]
            ridx[pl.ds(g * L, L)] = r16
            ee = (plsc.load_gather(wh12_v, [zero16i, r16])
                  + plsc.load_gather(wh12_v, [one16i, c16]))
            ee = jnp.where(ee >= 0, ee, ALPHA * ee)
            vv = jnp.exp(ee)
            v_v[pl.ds(g * L, L)] = vv
            plsc.addupdate_scatter(rs_v, [r16], vv)

        @plsc.parallel_loop(0, C, 1, unroll=8)
        def _(i):
            b = plsc.load_gather(v_v, [jnp.zeros((L,), jnp.int32) + i])
            for k in range(D // L):
                sl = (i, pl.ds(k * L, L))
                rows[sl] = rows[sl] * b

    @pl.loop(0, npairs)
    def _(i):
        c = start + 2 * i
        not_last = i < npairs - 1

        pltpu.make_async_copy(wh_hbm.at[cole_a], rows_a, gsem_a).wait()
        compute_v_and_scale(rowe_a, cole_a, ridx_a, rows_a)

        @pl.when(not_last)
        def _():
            pltpu.async_copy(erow_hbm.at[pl.ds((c + 2) * C, C)],
                             rowe_a, esem_a)
            pltpu.async_copy(ecol_hbm.at[pl.ds((c + 2) * C, C)],
                             cole_a, esem_a)

        pltpu.sync_copy(rows_a, unnorm_sh.at[ridx_a], add=True)

        @pl.when(not_last)
        def _():
            pltpu.make_async_copy(erow_hbm.at[pl.ds((c + 2) * C, C)],
                                  rowe_a, esem_a).wait()
            pltpu.make_async_copy(ecol_hbm.at[pl.ds((c + 2) * C, C)],
                                  cole_a, esem_a).wait()
            pltpu.async_copy(wh_hbm.at[cole_a], rows_a, gsem_a)

        pltpu.make_async_copy(wh_hbm.at[cole_b], rows_b, gsem_b).wait()
        compute_v_and_scale(rowe_b, cole_b, ridx_b, rows_b)

        @pl.when(not_last)
        def _():
            pltpu.async_copy(erow_hbm.at[pl.ds((c + 3) * C, C)],
                             rowe_b, esem_b)
            pltpu.async_copy(ecol_hbm.at[pl.ds((c + 3) * C, C)],
                             cole_b, esem_b)

        pltpu.sync_copy(rows_b, unnorm_sh.at[ridx_b], add=True)

        @pl.when(not_last)
        def _():
            pltpu.make_async_copy(erow_hbm.at[pl.ds((c + 3) * C, C)],
                                  rowe_b, esem_b).wait()
            pltpu.make_async_copy(ecol_hbm.at[pl.ds((c + 3) * C, C)],
                                  cole_b, esem_b).wait()
            pltpu.async_copy(wh_hbm.at[cole_b], rows_b, gsem_b)

    plsc.subcore_barrier()

    pltpu.sync_copy(unnorm_sh.at[pl.ds(sid * RPW, RPW)],
                    unnorm_hbm.at[cid, pl.ds(sid * RPW, RPW)])
    pltpu.sync_copy(rs_v, rowsum_hbm.at[pl.ds(wid * N_PAD, N_PAD)])


def _edge_pass(wh, wh12, erow, ecol):
    mesh = plsc.VectorSubcoreMesh(core_axis_name="c", subcore_axis_name="s")
    kern = pl.kernel(
        _edge_body,
        out_type=[
            jax.ShapeDtypeStruct((NC, N_PAD, D), jnp.float32),
            jax.ShapeDtypeStruct((NW * N_PAD,), jnp.float32),
        ],
        mesh=mesh,
        compiler_params=pltpu.CompilerParams(needs_layout_passes=False),
        scratch_types=[
            pltpu.VMEM((2, N), jnp.float32),
            pltpu.VMEM((N_PAD,), jnp.float32),
            pltpu.VMEM((C + L,), jnp.float32),
            pltpu.VMEM((C,), jnp.int32),
            pltpu.VMEM((C,), jnp.int32),
            pltpu.VMEM((C,), jnp.int32),
            pltpu.VMEM((C,), jnp.int32),
            pltpu.VMEM((C,), jnp.int32),
            pltpu.VMEM((C,), jnp.int32),
            pltpu.VMEM((C, D), jnp.float32),
            pltpu.VMEM((C, D), jnp.float32),
            pltpu.VMEM_SHARED((N_PAD, D), jnp.float32),
            pltpu.SemaphoreType.DMA,
            pltpu.SemaphoreType.DMA,
            pltpu.SemaphoreType.DMA,
            pltpu.SemaphoreType.DMA,
        ],
    )
    return kern(wh, wh12, erow, ecol)


def _combine_body(u_ref, rs_ref, out_ref):
    rs = jnp.sum(rs_ref[...], axis=0)
    u = u_ref[0] + u_ref[1]
    rs_col = rs[:, None]
    safe = jnp.where(rs_col > 0, rs_col, 1.0)
    x = jnp.where(rs_col > 0, u / safe, 0.0)
    out_ref[...] = jnp.where(x > 0, x, jnp.exp(jnp.minimum(x, 0.0)) - 1.0)


def _combine(unnorm, rowsum2d):
    return pl.pallas_call(
        _combine_body,
        out_shape=jax.ShapeDtypeStruct((N_PAD, D), jnp.float32),
    )(unnorm, rowsum2d)


def kernel(h, edge_index, W, a):
    a2 = jnp.concatenate([a[:D], a[D:]], axis=1)
    wh, wh12 = _dense(h, W, a2)
    unnorm, rowsum = _edge_pass(wh, wh12, edge_index[0], edge_index[1])
    out = _combine(unnorm, rowsum.reshape(NW, N_PAD))
    return out[:N]


# speedup vs baseline: 2.1238x; 1.0123x over previous
import jax
import jax.numpy as jnp
from jax import lax
from jax.experimental import pallas as pl
from jax.experimental.pallas import tpu as pltpu
from jax.experimental.pallas import tpu_sc as plsc

N = 10000
E = 320000
D = 128
ALPHA = 0.2

NC = 2
NS = 16
L = 16
NW = NC * NS
C = 64
NCH = E // C
BASE_PAIRS = NCH // 2 // NW
EXTRA = NCH // 2 - BASE_PAIRS * NW
N_PAD = 10112
RPW = N_PAD // NS


def _dense_body(h_ref, w_ref, a2_ref, wh_ref, wh12_ref):
    wh = jnp.dot(h_ref[...], w_ref[...],
                 preferred_element_type=jnp.float32,
                 precision=lax.Precision.HIGHEST)
    wh_ref[...] = wh
    wh12_ref[...] = lax.dot_general(
        a2_ref[...], wh, (((0,), (1,)), ((), ())),
        preferred_element_type=jnp.float32,
        precision=lax.Precision.HIGHEST)


def _dense(h, W, a2):
    return pl.pallas_call(
        _dense_body,
        out_shape=[
            jax.ShapeDtypeStruct((N, D), jnp.float32),
            jax.ShapeDtypeStruct((2, N), jnp.float32),
        ],
    )(h, W, a2)


def _edge_body(wh_hbm, wh12_hbm, erow_hbm, ecol_hbm, unnorm_hbm, rowsum_hbm,
               wh12_v, rs_v, v_v, rowe_a, cole_a, rowe_b, cole_b,
               ridx_a, ridx_b, rows_a, rows_b, unnorm_sh,
               gsem_a, gsem_b, esem_a, esem_b):
    cid = lax.axis_index("c")
    sid = lax.axis_index("s")
    wid = sid * NC + cid

    zeros = jnp.zeros((L,), jnp.float32)
    zero16i = jnp.zeros((L,), jnp.int32)
    one16i = jnp.ones((L,), jnp.int32)

    npairs = BASE_PAIRS + jnp.where(wid < EXTRA, 1, 0)
    start = 2 * (BASE_PAIRS * wid + lax.min(wid, EXTRA))

    @pl.loop(0, N_PAD // L)
    def _(i):
        rs_v[pl.ds(i * L, L)] = zeros

    @pl.loop(0, C)
    def _(i):
        for j in range(D // L):
            rows_a[i, pl.ds(j * L, L)] = zeros

    for k in range(RPW // C):
        pltpu.sync_copy(rows_a, unnorm_sh.at[pl.ds(sid * RPW + k * C, C)])
    pltpu.sync_copy(rows_a.at[pl.ds(0, RPW - (RPW // C) * C)],
                    unnorm_sh.at[pl.ds(sid * RPW + (RPW // C) * C,
                                       RPW - (RPW // C) * C)])

    pltpu.sync_copy(wh12_hbm, wh12_v)
    pltpu.sync_copy(erow_hbm.at[pl.ds(start * C, C)], rowe_a)
    pltpu.sync_copy(ecol_hbm.at[pl.ds(start * C, C)], cole_a)
    pltpu.sync_copy(erow_hbm.at[pl.ds((start + 1) * C, C)], rowe_b)
    pltpu.sync_copy(ecol_hbm.at[pl.ds((start + 1) * C, C)], cole_b)
    pltpu.async_copy(wh_hbm.at[cole_a], rows_a, gsem_a)
    pltpu.async_copy(wh_hbm.at[cole_b], rows_b, gsem_b)

    plsc.subcore_barrier()

    def compute_v_and_scale(rowe, cole, ridx, rows):
        for g in range(C // L):
            r16 = rowe[pl.ds(g * L, L)]
            c16 = cole[pl.ds(g * L, L)]
            ridx[pl.ds(g * L, L)] = r16
            ee = (plsc.load_gather(wh12_v, [zero16i, r16])
                  + plsc.load_gather(wh12_v, [one16i, c16]))
            ee = jnp.where(ee >= 0, ee, ALPHA * ee)
            vv = jnp.exp(ee)
            v_v[pl.ds(g * L, L)] = vv
            plsc.addupdate_scatter(rs_v, [r16], vv)

        @plsc.parallel_loop(0, C, 1, unroll=16)
        def _(i):
            b = plsc.load_gather(v_v, [jnp.zeros((L,), jnp.int32) + i])
            for k in range(D // L):
                sl = (i, pl.ds(k * L, L))
                rows[sl] = rows[sl] * b

    @pl.loop(0, npairs)
    def _(i):
        c = start + 2 * i
        not_last = i < npairs - 1

        pltpu.make_async_copy(wh_hbm.at[cole_a], rows_a, gsem_a).wait()
        compute_v_and_scale(rowe_a, cole_a, ridx_a, rows_a)

        @pl.when(not_last)
        def _():
            pltpu.async_copy(erow_hbm.at[pl.ds((c + 2) * C, C)],
                             rowe_a, esem_a)
            pltpu.async_copy(ecol_hbm.at[pl.ds((c + 2) * C, C)],
                             cole_a, esem_a)

        pltpu.sync_copy(rows_a, unnorm_sh.at[ridx_a], add=True)

        @pl.when(not_last)
        def _():
            pltpu.make_async_copy(erow_hbm.at[pl.ds((c + 2) * C, C)],
                                  rowe_a, esem_a).wait()
            pltpu.make_async_copy(ecol_hbm.at[pl.ds((c + 2) * C, C)],
                                  cole_a, esem_a).wait()
            pltpu.async_copy(wh_hbm.at[cole_a], rows_a, gsem_a)

        pltpu.make_async_copy(wh_hbm.at[cole_b], rows_b, gsem_b).wait()
        compute_v_and_scale(rowe_b, cole_b, ridx_b, rows_b)

        @pl.when(not_last)
        def _():
            pltpu.async_copy(erow_hbm.at[pl.ds((c + 3) * C, C)],
                             rowe_b, esem_b)
            pltpu.async_copy(ecol_hbm.at[pl.ds((c + 3) * C, C)],
                             cole_b, esem_b)

        pltpu.sync_copy(rows_b, unnorm_sh.at[ridx_b], add=True)

        @pl.when(not_last)
        def _():
            pltpu.make_async_copy(erow_hbm.at[pl.ds((c + 3) * C, C)],
                                  rowe_b, esem_b).wait()
            pltpu.make_async_copy(ecol_hbm.at[pl.ds((c + 3) * C, C)],
                                  cole_b, esem_b).wait()
            pltpu.async_copy(wh_hbm.at[cole_b], rows_b, gsem_b)

    plsc.subcore_barrier()

    pltpu.sync_copy(unnorm_sh.at[pl.ds(sid * RPW, RPW)],
                    unnorm_hbm.at[cid, pl.ds(sid * RPW, RPW)])
    pltpu.sync_copy(rs_v, rowsum_hbm.at[pl.ds(wid * N_PAD, N_PAD)])


def _edge_pass(wh, wh12, erow, ecol):
    mesh = plsc.VectorSubcoreMesh(core_axis_name="c", subcore_axis_name="s")
    kern = pl.kernel(
        _edge_body,
        out_type=[
            jax.ShapeDtypeStruct((NC, N_PAD, D), jnp.float32),
            jax.ShapeDtypeStruct((NW * N_PAD,), jnp.float32),
        ],
        mesh=mesh,
        compiler_params=pltpu.CompilerParams(needs_layout_passes=False),
        scratch_types=[
            pltpu.VMEM((2, N), jnp.float32),
            pltpu.VMEM((N_PAD,), jnp.float32),
            pltpu.VMEM((C + L,), jnp.float32),
            pltpu.VMEM((C,), jnp.int32),
            pltpu.VMEM((C,), jnp.int32),
            pltpu.VMEM((C,), jnp.int32),
            pltpu.VMEM((C,), jnp.int32),
            pltpu.VMEM((C,), jnp.int32),
            pltpu.VMEM((C,), jnp.int32),
            pltpu.VMEM((C, D), jnp.float32),
            pltpu.VMEM((C, D), jnp.float32),
            pltpu.VMEM_SHARED((N_PAD, D), jnp.float32),
            pltpu.SemaphoreType.DMA,
            pltpu.SemaphoreType.DMA,
            pltpu.SemaphoreType.DMA,
            pltpu.SemaphoreType.DMA,
        ],
    )
    return kern(wh, wh12, erow, ecol)


def _combine_body(u_ref, rs_ref, out_ref):
    rs = jnp.sum(rs_ref[...], axis=0)[:N]
    u = u_ref[0, :N] + u_ref[1, :N]
    rs_col = rs[:, None]
    safe = jnp.where(rs_col > 0, rs_col, 1.0)
    x = jnp.where(rs_col > 0, u / safe, 0.0)
    out_ref[...] = jnp.where(x > 0, x, jnp.exp(jnp.minimum(x, 0.0)) - 1.0)


def _combine(unnorm, rowsum2d):
    return pl.pallas_call(
        _combine_body,
        out_shape=jax.ShapeDtypeStruct((N, D), jnp.float32),
    )(unnorm, rowsum2d)


def kernel(h, edge_index, W, a):
    a2 = jnp.concatenate([a[:D], a[D:]], axis=1)
    wh, wh12 = _dense(h, W, a2)
    unnorm, rowsum = _edge_pass(wh, wh12, edge_index[0], edge_index[1])
    return _combine(unnorm, rowsum.reshape(NW, N_PAD))


# speedup vs baseline: 2.2171x; 1.0440x over previous
import jax
import jax.numpy as jnp
from jax import lax
from jax.experimental import pallas as pl
from jax.experimental.pallas import tpu as pltpu
from jax.experimental.pallas import tpu_sc as plsc

N = 10000
E = 320000
D = 128
ALPHA = 0.2

NC = 2
NS = 16
L = 16
NW = NC * NS
C = 64
NCH = E // C
BASE_PAIRS = NCH // 2 // NW
EXTRA = NCH // 2 - BASE_PAIRS * NW
N_PAD = 10112
RPW = N_PAD // NS


def _dense_body(h_ref, w_ref, a2_ref, wh_ref, wh12_ref):
    wh = jnp.dot(h_ref[...], w_ref[...],
                 preferred_element_type=jnp.float32,
                 precision=lax.Precision.DEFAULT)
    wh_ref[...] = wh
    wh12_ref[...] = lax.dot_general(
        a2_ref[...], wh, (((0,), (1,)), ((), ())),
        preferred_element_type=jnp.float32,
        precision=lax.Precision.HIGHEST)


def _dense(h, W, a2):
    return pl.pallas_call(
        _dense_body,
        out_shape=[
            jax.ShapeDtypeStruct((N, D), jnp.float32),
            jax.ShapeDtypeStruct((2, N), jnp.float32),
        ],
    )(h, W, a2)


def _edge_body(wh_hbm, wh12_hbm, erow_hbm, ecol_hbm, unnorm_hbm, rowsum_hbm,
               wh12_v, rs_v, v_v, rowe_a, cole_a, rowe_b, cole_b,
               ridx_a, ridx_b, rows_a, rows_b, unnorm_sh,
               gsem_a, gsem_b, esem_a, esem_b):
    cid = lax.axis_index("c")
    sid = lax.axis_index("s")
    wid = sid * NC + cid

    zeros = jnp.zeros((L,), jnp.float32)
    zero16i = jnp.zeros((L,), jnp.int32)
    one16i = jnp.ones((L,), jnp.int32)

    npairs = BASE_PAIRS + jnp.where(wid < EXTRA, 1, 0)
    start = 2 * (BASE_PAIRS * wid + lax.min(wid, EXTRA))

    @pl.loop(0, N_PAD // L)
    def _(i):
        rs_v[pl.ds(i * L, L)] = zeros

    @pl.loop(0, C)
    def _(i):
        for j in range(D // L):
            rows_a[i, pl.ds(j * L, L)] = zeros

    for k in range(RPW // C):
        pltpu.sync_copy(rows_a, unnorm_sh.at[pl.ds(sid * RPW + k * C, C)])
    pltpu.sync_copy(rows_a.at[pl.ds(0, RPW - (RPW // C) * C)],
                    unnorm_sh.at[pl.ds(sid * RPW + (RPW // C) * C,
                                       RPW - (RPW // C) * C)])

    pltpu.sync_copy(wh12_hbm, wh12_v)
    pltpu.sync_copy(erow_hbm.at[pl.ds(start * C, C)], rowe_a)
    pltpu.sync_copy(ecol_hbm.at[pl.ds(start * C, C)], cole_a)
    pltpu.sync_copy(erow_hbm.at[pl.ds((start + 1) * C, C)], rowe_b)
    pltpu.sync_copy(ecol_hbm.at[pl.ds((start + 1) * C, C)], cole_b)
    pltpu.async_copy(wh_hbm.at[cole_a], rows_a, gsem_a)
    pltpu.async_copy(wh_hbm.at[cole_b], rows_b, gsem_b)

    plsc.subcore_barrier()

    def compute_v_and_scale(rowe, cole, ridx, rows):
        for g in range(C // L):
            r16 = rowe[pl.ds(g * L, L)]
            c16 = cole[pl.ds(g * L, L)]
            ridx[pl.ds(g * L, L)] = r16
            ee = (plsc.load_gather(wh12_v, [zero16i, r16])
                  + plsc.load_gather(wh12_v, [one16i, c16]))
            ee = jnp.where(ee >= 0, ee, ALPHA * ee)
            vv = jnp.exp(ee)
            v_v[pl.ds(g * L, L)] = vv
            plsc.addupdate_scatter(rs_v, [r16], vv)

        @plsc.parallel_loop(0, C, 1, unroll=16)
        def _(i):
            b = plsc.load_gather(v_v, [jnp.zeros((L,), jnp.int32) + i])
            for k in range(D // L):
                sl = (i, pl.ds(k * L, L))
                rows[sl] = rows[sl] * b

    @pl.loop(0, npairs)
    def _(i):
        c = start + 2 * i
        not_last = i < npairs - 1

        pltpu.make_async_copy(wh_hbm.at[cole_a], rows_a, gsem_a).wait()
        compute_v_and_scale(rowe_a, cole_a, ridx_a, rows_a)

        @pl.when(not_last)
        def _():
            pltpu.async_copy(erow_hbm.at[pl.ds((c + 2) * C, C)],
                             rowe_a, esem_a)
            pltpu.async_copy(ecol_hbm.at[pl.ds((c + 2) * C, C)],
                             cole_a, esem_a)

        pltpu.sync_copy(rows_a, unnorm_sh.at[ridx_a], add=True)

        @pl.when(not_last)
        def _():
            pltpu.make_async_copy(erow_hbm.at[pl.ds((c + 2) * C, C)],
                                  rowe_a, esem_a).wait()
            pltpu.make_async_copy(ecol_hbm.at[pl.ds((c + 2) * C, C)],
                                  cole_a, esem_a).wait()
            pltpu.async_copy(wh_hbm.at[cole_a], rows_a, gsem_a)

        pltpu.make_async_copy(wh_hbm.at[cole_b], rows_b, gsem_b).wait()
        compute_v_and_scale(rowe_b, cole_b, ridx_b, rows_b)

        @pl.when(not_last)
        def _():
            pltpu.async_copy(erow_hbm.at[pl.ds((c + 3) * C, C)],
                             rowe_b, esem_b)
            pltpu.async_copy(ecol_hbm.at[pl.ds((c + 3) * C, C)],
                             cole_b, esem_b)

        pltpu.sync_copy(rows_b, unnorm_sh.at[ridx_b], add=True)

        @pl.when(not_last)
        def _():
            pltpu.make_async_copy(erow_hbm.at[pl.ds((c + 3) * C, C)],
                                  rowe_b, esem_b).wait()
            pltpu.make_async_copy(ecol_hbm.at[pl.ds((c + 3) * C, C)],
                                  cole_b, esem_b).wait()
            pltpu.async_copy(wh_hbm.at[cole_b], rows_b, gsem_b)

    plsc.subcore_barrier()

    pltpu.sync_copy(unnorm_sh.at[pl.ds(sid * RPW, RPW)],
                    unnorm_hbm.at[cid, pl.ds(sid * RPW, RPW)])
    pltpu.sync_copy(rs_v, rowsum_hbm.at[pl.ds(wid * N_PAD, N_PAD)])


def _edge_pass(wh, wh12, erow, ecol):
    mesh = plsc.VectorSubcoreMesh(core_axis_name="c", subcore_axis_name="s")
    kern = pl.kernel(
        _edge_body,
        out_type=[
            jax.ShapeDtypeStruct((NC, N_PAD, D), jnp.float32),
            jax.ShapeDtypeStruct((NW * N_PAD,), jnp.float32),
        ],
        mesh=mesh,
        compiler_params=pltpu.CompilerParams(needs_layout_passes=False),
        scratch_types=[
            pltpu.VMEM((2, N), jnp.float32),
            pltpu.VMEM((N_PAD,), jnp.float32),
            pltpu.VMEM((C + L,), jnp.float32),
            pltpu.VMEM((C,), jnp.int32),
            pltpu.VMEM((C,), jnp.int32),
            pltpu.VMEM((C,), jnp.int32),
            pltpu.VMEM((C,), jnp.int32),
            pltpu.VMEM((C,), jnp.int32),
            pltpu.VMEM((C,), jnp.int32),
            pltpu.VMEM((C, D), jnp.float32),
            pltpu.VMEM((C, D), jnp.float32),
            pltpu.VMEM_SHARED((N_PAD, D), jnp.float32),
            pltpu.SemaphoreType.DMA,
            pltpu.SemaphoreType.DMA,
            pltpu.SemaphoreType.DMA,
            pltpu.SemaphoreType.DMA,
        ],
    )
    return kern(wh, wh12, erow, ecol)


def _combine_body(u_ref, rs_ref, out_ref):
    rs = jnp.sum(rs_ref[...], axis=0)[:N]
    u = u_ref[0, :N] + u_ref[1, :N]
    rs_col = rs[:, None]
    safe = jnp.where(rs_col > 0, rs_col, 1.0)
    x = jnp.where(rs_col > 0, u / safe, 0.0)
    out_ref[...] = jnp.where(x > 0, x, jnp.exp(jnp.minimum(x, 0.0)) - 1.0)


def _combine(unnorm, rowsum2d):
    return pl.pallas_call(
        _combine_body,
        out_shape=jax.ShapeDtypeStruct((N, D), jnp.float32),
    )(unnorm, rowsum2d)


def kernel(h, edge_index, W, a):
    a2 = jnp.concatenate([a[:D], a[D:]], axis=1)
    wh, wh12 = _dense(h, W, a2)
    unnorm, rowsum = _edge_pass(wh, wh12, edge_index[0], edge_index[1])
    return _combine(unnorm, rowsum.reshape(NW, N_PAD))


# speedup vs baseline: 2.2341x; 1.0077x over previous
import jax
import jax.numpy as jnp
from jax import lax
from jax.experimental import pallas as pl
from jax.experimental.pallas import tpu as pltpu
from jax.experimental.pallas import tpu_sc as plsc

N = 10000
E = 320000
D = 128
ALPHA = 0.2

NC = 2
NS = 16
L = 16
NW = NC * NS
C = 64
NCH = E // C
BASE_PAIRS = NCH // 2 // NW
EXTRA = NCH // 2 - BASE_PAIRS * NW
N_PAD = 10112
RPW = N_PAD // NS


def _dense_body(h_ref, w_ref, a2_ref, wh_ref, wh12_ref):
    wh = jnp.dot(h_ref[...], w_ref[...],
                 preferred_element_type=jnp.float32,
                 precision=lax.Precision.DEFAULT)
    wh_ref[...] = wh
    wh12_ref[...] = lax.dot_general(
        a2_ref[...], wh, (((0,), (1,)), ((), ())),
        preferred_element_type=jnp.float32,
        precision=lax.Precision.DEFAULT)


def _dense(h, W, a2):
    return pl.pallas_call(
        _dense_body,
        out_shape=[
            jax.ShapeDtypeStruct((N, D), jnp.float32),
            jax.ShapeDtypeStruct((2, N), jnp.float32),
        ],
    )(h, W, a2)


def _edge_body(wh_hbm, wh12_hbm, erow_hbm, ecol_hbm, unnorm_hbm, rowsum_hbm,
               wh12_v, rs_v, v_v, rowe_a, cole_a, rowe_b, cole_b,
               ridx_a, ridx_b, rows_a, rows_b, unnorm_sh,
               gsem_a, gsem_b, esem_a, esem_b):
    cid = lax.axis_index("c")
    sid = lax.axis_index("s")
    wid = sid * NC + cid

    zeros = jnp.zeros((L,), jnp.float32)
    zero16i = jnp.zeros((L,), jnp.int32)
    one16i = jnp.ones((L,), jnp.int32)

    npairs = BASE_PAIRS + jnp.where(wid < EXTRA, 1, 0)
    start = 2 * (BASE_PAIRS * wid + lax.min(wid, EXTRA))

    @pl.loop(0, N_PAD // L)
    def _(i):
        rs_v[pl.ds(i * L, L)] = zeros

    @pl.loop(0, C)
    def _(i):
        for j in range(D // L):
            rows_a[i, pl.ds(j * L, L)] = zeros

    for k in range(RPW // C):
        pltpu.sync_copy(rows_a, unnorm_sh.at[pl.ds(sid * RPW + k * C, C)])
    pltpu.sync_copy(rows_a.at[pl.ds(0, RPW - (RPW // C) * C)],
                    unnorm_sh.at[pl.ds(sid * RPW + (RPW // C) * C,
                                       RPW - (RPW // C) * C)])

    pltpu.sync_copy(wh12_hbm, wh12_v)
    pltpu.sync_copy(erow_hbm.at[pl.ds(start * C, C)], rowe_a)
    pltpu.sync_copy(ecol_hbm.at[pl.ds(start * C, C)], cole_a)
    pltpu.sync_copy(erow_hbm.at[pl.ds((start + 1) * C, C)], rowe_b)
    pltpu.sync_copy(ecol_hbm.at[pl.ds((start + 1) * C, C)], cole_b)
    pltpu.async_copy(wh_hbm.at[cole_a], rows_a, gsem_a)
    pltpu.async_copy(wh_hbm.at[cole_b], rows_b, gsem_b)

    plsc.subcore_barrier()

    def compute_v_and_scale(rowe, cole, ridx, rows):
        for g in range(C // L):
            r16 = rowe[pl.ds(g * L, L)]
            c16 = cole[pl.ds(g * L, L)]
            ridx[pl.ds(g * L, L)] = r16
            ee = (plsc.load_gather(wh12_v, [zero16i, r16])
                  + plsc.load_gather(wh12_v, [one16i, c16]))
            ee = jnp.where(ee >= 0, ee, ALPHA * ee)
            vv = jnp.exp(ee)
            v_v[pl.ds(g * L, L)] = vv
            plsc.addupdate_scatter(rs_v, [r16], vv)

        @plsc.parallel_loop(0, C, 1, unroll=16)
        def _(i):
            b = plsc.load_gather(v_v, [jnp.zeros((L,), jnp.int32) + i])
            for k in range(D // L):
                sl = (i, pl.ds(k * L, L))
                rows[sl] = rows[sl] * b

    @pl.loop(0, npairs)
    def _(i):
        c = start + 2 * i
        not_last = i < npairs - 1

        pltpu.make_async_copy(wh_hbm.at[cole_a], rows_a, gsem_a).wait()
        compute_v_and_scale(rowe_a, cole_a, ridx_a, rows_a)

        @pl.when(not_last)
        def _():
            pltpu.async_copy(erow_hbm.at[pl.ds((c + 2) * C, C)],
                             rowe_a, esem_a)
            pltpu.async_copy(ecol_hbm.at[pl.ds((c + 2) * C, C)],
                             cole_a, esem_a)

        pltpu.sync_copy(rows_a, unnorm_sh.at[ridx_a], add=True)

        @pl.when(not_last)
        def _():
            pltpu.make_async_copy(erow_hbm.at[pl.ds((c + 2) * C, C)],
                                  rowe_a, esem_a).wait()
            pltpu.make_async_copy(ecol_hbm.at[pl.ds((c + 2) * C, C)],
                                  cole_a, esem_a).wait()
            pltpu.async_copy(wh_hbm.at[cole_a], rows_a, gsem_a)

        pltpu.make_async_copy(wh_hbm.at[cole_b], rows_b, gsem_b).wait()
        compute_v_and_scale(rowe_b, cole_b, ridx_b, rows_b)

        @pl.when(not_last)
        def _():
            pltpu.async_copy(erow_hbm.at[pl.ds((c + 3) * C, C)],
                             rowe_b, esem_b)
            pltpu.async_copy(ecol_hbm.at[pl.ds((c + 3) * C, C)],
                             cole_b, esem_b)

        pltpu.sync_copy(rows_b, unnorm_sh.at[ridx_b], add=True)

        @pl.when(not_last)
        def _():
            pltpu.make_async_copy(erow_hbm.at[pl.ds((c + 3) * C, C)],
                                  rowe_b, esem_b).wait()
            pltpu.make_async_copy(ecol_hbm.at[pl.ds((c + 3) * C, C)],
                                  cole_b, esem_b).wait()
            pltpu.async_copy(wh_hbm.at[cole_b], rows_b, gsem_b)

    plsc.subcore_barrier()

    pltpu.sync_copy(unnorm_sh.at[pl.ds(sid * RPW, RPW)],
                    unnorm_hbm.at[cid, pl.ds(sid * RPW, RPW)])
    pltpu.sync_copy(rs_v, rowsum_hbm.at[pl.ds(wid * N_PAD, N_PAD)])


def _edge_pass(wh, wh12, erow, ecol):
    mesh = plsc.VectorSubcoreMesh(core_axis_name="c", subcore_axis_name="s")
    kern = pl.kernel(
        _edge_body,
        out_type=[
            jax.ShapeDtypeStruct((NC, N_PAD, D), jnp.float32),
            jax.ShapeDtypeStruct((NW * N_PAD,), jnp.float32),
        ],
        mesh=mesh,
        compiler_params=pltpu.CompilerParams(needs_layout_passes=False),
        scratch_types=[
            pltpu.VMEM((2, N), jnp.float32),
            pltpu.VMEM((N_PAD,), jnp.float32),
            pltpu.VMEM((C + L,), jnp.float32),
            pltpu.VMEM((C,), jnp.int32),
            pltpu.VMEM((C,), jnp.int32),
            pltpu.VMEM((C,), jnp.int32),
            pltpu.VMEM((C,), jnp.int32),
            pltpu.VMEM((C,), jnp.int32),
            pltpu.VMEM((C,), jnp.int32),
            pltpu.VMEM((C, D), jnp.float32),
            pltpu.VMEM((C, D), jnp.float32),
            pltpu.VMEM_SHARED((N_PAD, D), jnp.float32),
            pltpu.SemaphoreType.DMA,
            pltpu.SemaphoreType.DMA,
            pltpu.SemaphoreType.DMA,
            pltpu.SemaphoreType.DMA,
        ],
    )
    return kern(wh, wh12, erow, ecol)


def _combine_body(u_ref, rs_ref, out_ref):
    rs = jnp.sum(rs_ref[...], axis=0)[:N]
    u = u_ref[0, :N] + u_ref[1, :N]
    rs_col = rs[:, None]
    safe = jnp.where(rs_col > 0, rs_col, 1.0)
    x = jnp.where(rs_col > 0, u / safe, 0.0)
    out_ref[...] = jnp.where(x > 0, x, jnp.exp(jnp.minimum(x, 0.0)) - 1.0)


def _combine(unnorm, rowsum2d):
    return pl.pallas_call(
        _combine_body,
        out_shape=jax.ShapeDtypeStruct((N, D), jnp.float32),
    )(unnorm, rowsum2d)


def kernel(h, edge_index, W, a):
    a2 = jnp.concatenate([a[:D], a[D:]], axis=1)
    wh, wh12 = _dense(h, W, a2)
    unnorm, rowsum = _edge_pass(wh, wh12, edge_index[0], edge_index[1])
    return _combine(unnorm, rowsum.reshape(NW, N_PAD))


# speedup vs baseline: 2.2851x; 1.0228x over previous
import jax
import jax.numpy as jnp
from jax import lax
from jax.experimental import pallas as pl
from jax.experimental.pallas import tpu as pltpu
from jax.experimental.pallas import tpu_sc as plsc

N = 10000
E = 320000
D = 128
ALPHA = 0.2

NC = 2
NS = 16
L = 16
NW = NC * NS
C = 64
NCH = E // C
BASE_PAIRS = NCH // 2 // NW
EXTRA = NCH // 2 - BASE_PAIRS * NW
N_PAD = 10112
RPW = N_PAD // NS


def _dense_body(h_ref, w_ref, a2_ref, wh_ref, wh12_ref):
    wh = jnp.dot(h_ref[...], w_ref[...],
                 preferred_element_type=jnp.float32,
                 precision=lax.Precision.DEFAULT)
    wh_ref[...] = wh
    wh12_ref[...] = lax.dot_general(
        a2_ref[...], wh, (((0,), (1,)), ((), ())),
        preferred_element_type=jnp.float32,
        precision=lax.Precision.DEFAULT)


def _dense(h, W, a2):
    return pl.pallas_call(
        _dense_body,
        out_shape=[
            jax.ShapeDtypeStruct((N, D), jnp.float32),
            jax.ShapeDtypeStruct((2, N), jnp.float32),
        ],
    )(h, W, a2)


def _edge_body(wh_hbm, wh12_hbm, erow_hbm, ecol_hbm, unnorm_hbm, rowsum_hbm,
               wh12_v, rs_v, v_v, rowe_a, cole_a, rowe_b, cole_b,
               ridx_a, ridx_b, rows_a, rows_b, unnorm_sh,
               gsem_a, gsem_b, esem_a, esem_b):
    cid = lax.axis_index("c")
    sid = lax.axis_index("s")
    wid = sid * NC + cid

    zeros = jnp.zeros((L,), jnp.float32)
    zero16i = jnp.zeros((L,), jnp.int32)
    one16i = jnp.ones((L,), jnp.int32)

    npairs = BASE_PAIRS + jnp.where(wid < EXTRA, 1, 0)
    start = 2 * (BASE_PAIRS * wid + lax.min(wid, EXTRA))

    @pl.loop(0, N_PAD // L)
    def _(i):
        rs_v[pl.ds(i * L, L)] = zeros

    @pl.loop(0, C)
    def _(i):
        for j in range(D // L):
            rows_a[i, pl.ds(j * L, L)] = zeros

    for k in range(RPW // C):
        pltpu.sync_copy(rows_a, unnorm_sh.at[pl.ds(sid * RPW + k * C, C)])
    pltpu.sync_copy(rows_a.at[pl.ds(0, RPW - (RPW // C) * C)],
                    unnorm_sh.at[pl.ds(sid * RPW + (RPW // C) * C,
                                       RPW - (RPW // C) * C)])

    pltpu.sync_copy(wh12_hbm, wh12_v)
    pltpu.sync_copy(erow_hbm.at[pl.ds(start * C, C)], rowe_a)
    pltpu.sync_copy(ecol_hbm.at[pl.ds(start * C, C)], cole_a)
    pltpu.sync_copy(erow_hbm.at[pl.ds((start + 1) * C, C)], rowe_b)
    pltpu.sync_copy(ecol_hbm.at[pl.ds((start + 1) * C, C)], cole_b)
    pltpu.async_copy(wh_hbm.at[cole_a], rows_a, gsem_a)
    pltpu.async_copy(wh_hbm.at[cole_b], rows_b, gsem_b)

    plsc.subcore_barrier()

    def compute_v_and_scale(rowe, cole, ridx, rows):
        @plsc.parallel_loop(0, C // L, 1, unroll=C // L)
        def _(g):
            r16 = rowe[pl.ds(g * L, L)]
            c16 = cole[pl.ds(g * L, L)]
            ridx[pl.ds(g * L, L)] = r16
            ee = (plsc.load_gather(wh12_v, [zero16i, r16])
                  + plsc.load_gather(wh12_v, [one16i, c16]))
            ee = jnp.where(ee >= 0, ee, ALPHA * ee)
            vv = jnp.exp(ee)
            v_v[pl.ds(g * L, L)] = vv
            plsc.addupdate_scatter(rs_v, [r16], vv)

        @plsc.parallel_loop(0, C, 1, unroll=16)
        def _(i):
            b = plsc.load_gather(v_v, [jnp.zeros((L,), jnp.int32) + i])
            for k in range(D // L):
                sl = (i, pl.ds(k * L, L))
                rows[sl] = rows[sl] * b

    @pl.loop(0, npairs)
    def _(i):
        c = start + 2 * i
        not_last = i < npairs - 1

        pltpu.make_async_copy(wh_hbm.at[cole_a], rows_a, gsem_a).wait()
        compute_v_and_scale(rowe_a, cole_a, ridx_a, rows_a)

        @pl.when(not_last)
        def _():
            pltpu.async_copy(erow_hbm.at[pl.ds((c + 2) * C, C)],
                             rowe_a, esem_a)
            pltpu.async_copy(ecol_hbm.at[pl.ds((c + 2) * C, C)],
                             cole_a, esem_a)

        pltpu.sync_copy(rows_a, unnorm_sh.at[ridx_a], add=True)

        @pl.when(not_last)
        def _():
            pltpu.make_async_copy(erow_hbm.at[pl.ds((c + 2) * C, C)],
                                  rowe_a, esem_a).wait()
            pltpu.make_async_copy(ecol_hbm.at[pl.ds((c + 2) * C, C)],
                                  cole_a, esem_a).wait()
            pltpu.async_copy(wh_hbm.at[cole_a], rows_a, gsem_a)

        pltpu.make_async_copy(wh_hbm.at[cole_b], rows_b, gsem_b).wait()
        compute_v_and_scale(rowe_b, cole_b, ridx_b, rows_b)

        @pl.when(not_last)
        def _():
            pltpu.async_copy(erow_hbm.at[pl.ds((c + 3) * C, C)],
                             rowe_b, esem_b)
            pltpu.async_copy(ecol_hbm.at[pl.ds((c + 3) * C, C)],
                             cole_b, esem_b)

        pltpu.sync_copy(rows_b, unnorm_sh.at[ridx_b], add=True)

        @pl.when(not_last)
        def _():
            pltpu.make_async_copy(erow_hbm.at[pl.ds((c + 3) * C, C)],
                                  rowe_b, esem_b).wait()
            pltpu.make_async_copy(ecol_hbm.at[pl.ds((c + 3) * C, C)],
                                  cole_b, esem_b).wait()
            pltpu.async_copy(wh_hbm.at[cole_b], rows_b, gsem_b)

    plsc.subcore_barrier()

    pltpu.sync_copy(unnorm_sh.at[pl.ds(sid * RPW, RPW)],
                    unnorm_hbm.at[cid, pl.ds(sid * RPW, RPW)])
    pltpu.sync_copy(rs_v, rowsum_hbm.at[pl.ds(wid * N_PAD, N_PAD)])


def _edge_pass(wh, wh12, erow, ecol):
    mesh = plsc.VectorSubcoreMesh(core_axis_name="c", subcore_axis_name="s")
    kern = pl.kernel(
        _edge_body,
        out_type=[
            jax.ShapeDtypeStruct((NC, N_PAD, D), jnp.float32),
            jax.ShapeDtypeStruct((NW * N_PAD,), jnp.float32),
        ],
        mesh=mesh,
        compiler_params=pltpu.CompilerParams(needs_layout_passes=False),
        scratch_types=[
            pltpu.VMEM((2, N), jnp.float32),
            pltpu.VMEM((N_PAD,), jnp.float32),
            pltpu.VMEM((C + L,), jnp.float32),
            pltpu.VMEM((C,), jnp.int32),
            pltpu.VMEM((C,), jnp.int32),
            pltpu.VMEM((C,), jnp.int32),
            pltpu.VMEM((C,), jnp.int32),
            pltpu.VMEM((C,), jnp.int32),
            pltpu.VMEM((C,), jnp.int32),
            pltpu.VMEM((C, D), jnp.float32),
            pltpu.VMEM((C, D), jnp.float32),
            pltpu.VMEM_SHARED((N_PAD, D), jnp.float32),
            pltpu.SemaphoreType.DMA,
            pltpu.SemaphoreType.DMA,
            pltpu.SemaphoreType.DMA,
            pltpu.SemaphoreType.DMA,
        ],
    )
    return kern(wh, wh12, erow, ecol)


def _combine_body(u_ref, rs_ref, out_ref):
    rs = jnp.sum(rs_ref[...], axis=0)[:N]
    u = u_ref[0, :N] + u_ref[1, :N]
    rs_col = rs[:, None]
    safe = jnp.where(rs_col > 0, rs_col, 1.0)
    x = jnp.where(rs_col > 0, u / safe, 0.0)
    out_ref[...] = jnp.where(x > 0, x, jnp.exp(jnp.minimum(x, 0.0)) - 1.0)


def _combine(unnorm, rowsum2d):
    return pl.pallas_call(
        _combine_body,
        out_shape=jax.ShapeDtypeStruct((N, D), jnp.float32),
    )(unnorm, rowsum2d)


def kernel(h, edge_index, W, a):
    a2 = jnp.concatenate([a[:D], a[D:]], axis=1)
    wh, wh12 = _dense(h, W, a2)
    unnorm, rowsum = _edge_pass(wh, wh12, edge_index[0], edge_index[1])
    return _combine(unnorm, rowsum.reshape(NW, N_PAD))
